# baseline (device time: 35464 ns/iter reference)
import jax
import jax.numpy as jnp
from jax import lax
from jax.experimental import pallas as pl
from jax.experimental.pallas import tpu as pltpu

N_DEV = 16
EPS = 1e-5
GLOBAL_HW = 2048 * 128
NCHUNK = 8


def kernel(x, Wp):
    b, h_loc, w, c = x.shape
    c_out = Wp.shape[1]
    rows = h_loc * w
    ch = rows // NCHUNK
    x3 = x.reshape(b, rows, c)

    def body(
        x_hbm, wp_ref, out_hbm,
        xbuf, obuf, comm_ref,
        load_sems, store_sems, send_sems, recv_sems,
    ):
        my_pos = lax.axis_index("i")

        barrier_sem = pltpu.get_barrier_semaphore()
        for d in range(1, N_DEV):
            pl.semaphore_signal(
                barrier_sem,
                inc=1,
                device_id=((my_pos + d) % N_DEV,),
                device_id_type=pl.DeviceIdType.MESH,
            )

        loads = []
        for k in range(NCHUNK):
            cp = pltpu.make_async_copy(
                x_hbm.at[:, pl.ds(k * ch, ch), :],
                xbuf.at[:, pl.ds(k * ch, ch), :],
                load_sems.at[k],
            )
            cp.start()
            loads.append(cp)
        s = jnp.zeros((b, c), jnp.float32)
        ss = jnp.zeros((b, c), jnp.float32)
        for k in range(NCHUNK):
            loads[k].wait()
            xk = xbuf[:, pl.ds(k * ch, ch), :]
            s = s + jnp.sum(xk, axis=1)
            ss = ss + jnp.sum(xk * xk, axis=1)
        comm_ref[0, 0:b, :] = s
        comm_ref[0, b : 2 * b, :] = ss

        pl.semaphore_wait(barrier_sem, N_DEV - 1)

        rdmas = []
        for d in range(1, N_DEV):
            target = (my_pos + d) % N_DEV
            rdma = pltpu.make_async_remote_copy(
                src_ref=comm_ref.at[0],
                dst_ref=comm_ref.at[d],
                send_sem=send_sems.at[d],
                recv_sem=recv_sems.at[d],
                device_id=(target,),
                device_id_type=pl.DeviceIdType.MESH,
            )
            rdma.start()
            rdmas.append(rdma)
        for rdma in rdmas:
            rdma.wait_recv()

        total = jnp.sum(comm_ref[...], axis=0)
        mean = total[0:b, :] / GLOBAL_HW
        ex2 = total[b : 2 * b, :] / GLOBAL_HW
        inv = lax.rsqrt(ex2 - mean * mean + EPS)
        shift = -mean * inv
        wp = wp_ref[...].astype(jnp.bfloat16)

        stores = []
        for k in range(NCHUNK):
            slot = k % 2
            if k >= 2:
                stores[k - 2].wait()
            xk = xbuf[:, pl.ds(k * ch, ch), :]
            h = (xk * inv[:, None, :] + shift[:, None, :]).astype(jnp.bfloat16)
            a = (jnp.bfloat16(0.5) * h) * (
                jnp.tanh(jnp.bfloat16(0.5) * h) + jnp.bfloat16(1.0)
            )
            for bb in range(b):
                obuf[slot, bb] = jnp.dot(
                    a[bb], wp, preferred_element_type=jnp.float32
                ).astype(obuf.dtype)
            st = pltpu.make_async_copy(
                obuf.at[slot],
                out_hbm.at[:, pl.ds(k * ch, ch), :],
                store_sems.at[slot],
            )
            st.start()
            stores.append(st)
        stores[NCHUNK - 2].wait()
        stores[NCHUNK - 1].wait()

        for rdma in rdmas:
            rdma.wait_send()

    out = pl.pallas_call(
        body,
        out_shape=jax.ShapeDtypeStruct((b, rows, c_out), jnp.bfloat16),
        in_specs=[
            pl.BlockSpec(memory_space=pl.ANY),
            pl.BlockSpec(memory_space=pltpu.VMEM),
        ],
        out_specs=pl.BlockSpec(memory_space=pl.ANY),
        scratch_shapes=[
            pltpu.VMEM((b, rows, c), jnp.float32),
            pltpu.VMEM((2, b, rows // NCHUNK, c_out), jnp.bfloat16),
            pltpu.VMEM((N_DEV, 2 * b, c), jnp.float32),
            pltpu.SemaphoreType.DMA((NCHUNK,)),
            pltpu.SemaphoreType.DMA((2,)),
            pltpu.SemaphoreType.DMA((N_DEV,)),
            pltpu.SemaphoreType.DMA((N_DEV,)),
        ],
        compiler_params=pltpu.CompilerParams(collective_id=0),
    )(x3, Wp)
    return out.reshape(b, h_loc, w, c_out)


# device time: 34688 ns/iter; 1.0224x vs baseline; 1.0224x over previous
import jax
import jax.numpy as jnp
from jax import lax
from jax.experimental import pallas as pl
from jax.experimental.pallas import tpu as pltpu

N_DEV = 16
EPS = 1e-5
GLOBAL_HW = 2048 * 128
NCHUNK = 4


def kernel(x, Wp):
    b, h_loc, w, c = x.shape
    c_out = Wp.shape[1]
    rows = h_loc * w
    ch = rows // NCHUNK
    x3 = x.reshape(b, rows, c)

    def body(
        x_hbm, wp_ref, out_hbm,
        xbuf, obuf, comm_ref,
        load_sems, store_sems, send_sems, recv_sems,
    ):
        my_pos = lax.axis_index("i")

        barrier_sem = pltpu.get_barrier_semaphore()
        for d in range(1, N_DEV):
            pl.semaphore_signal(
                barrier_sem,
                inc=1,
                device_id=((my_pos + d) % N_DEV,),
                device_id_type=pl.DeviceIdType.MESH,
            )

        loads = []
        for k in range(NCHUNK):
            cp = pltpu.make_async_copy(
                x_hbm.at[:, pl.ds(k * ch, ch), :],
                xbuf.at[:, pl.ds(k * ch, ch), :],
                load_sems.at[k],
            )
            cp.start()
            loads.append(cp)
        ones_row = jnp.ones((1, ch), jnp.float32)
        s = [jnp.zeros((1, c), jnp.float32) for _ in range(b)]
        ss = [jnp.zeros((1, c), jnp.float32) for _ in range(b)]
        for k in range(NCHUNK):
            loads[k].wait()
            xk = xbuf[:, pl.ds(k * ch, ch), :]
            sq = xk * xk
            for bb in range(b):
                s[bb] = s[bb] + jnp.dot(
                    ones_row, xk[bb], preferred_element_type=jnp.float32
                )
                ss[bb] = ss[bb] + jnp.dot(
                    ones_row, sq[bb], preferred_element_type=jnp.float32
                )
        comm_ref[0, 0:b, :] = jnp.concatenate(s, axis=0)
        comm_ref[0, b : 2 * b, :] = jnp.concatenate(ss, axis=0)

        pl.semaphore_wait(barrier_sem, N_DEV - 1)

        rdmas = []
        for d in range(1, N_DEV):
            target = (my_pos + d) % N_DEV
            rdma = pltpu.make_async_remote_copy(
                src_ref=comm_ref.at[0],
                dst_ref=comm_ref.at[d],
                send_sem=send_sems.at[d],
                recv_sem=recv_sems.at[d],
                device_id=(target,),
                device_id_type=pl.DeviceIdType.MESH,
            )
            rdma.start()
            rdmas.append(rdma)
        for rdma in rdmas:
            rdma.wait_recv()

        total = jnp.sum(comm_ref[...], axis=0)
        mean = total[0:b, :] / GLOBAL_HW
        ex2 = total[b : 2 * b, :] / GLOBAL_HW
        inv = lax.rsqrt(ex2 - mean * mean + EPS)
        shift = -mean * inv
        wp = wp_ref[...].astype(jnp.bfloat16)

        stores = []
        for k in range(NCHUNK):
            slot = k % 2
            if k >= 2:
                stores[k - 2].wait()
            xk = xbuf[:, pl.ds(k * ch, ch), :]
            h = (xk * inv[:, None, :] + shift[:, None, :]).astype(jnp.bfloat16)
            a = (jnp.bfloat16(0.5) * h) * (
                jnp.tanh(jnp.bfloat16(0.5) * h) + jnp.bfloat16(1.0)
            )
            for bb in range(b):
                obuf[slot, bb] = jnp.dot(
                    a[bb], wp, preferred_element_type=jnp.float32
                ).astype(obuf.dtype)
            st = pltpu.make_async_copy(
                obuf.at[slot],
                out_hbm.at[:, pl.ds(k * ch, ch), :],
                store_sems.at[slot],
            )
            st.start()
            stores.append(st)
        stores[NCHUNK - 2].wait()
        stores[NCHUNK - 1].wait()

        for rdma in rdmas:
            rdma.wait_send()

    out = pl.pallas_call(
        body,
        out_shape=jax.ShapeDtypeStruct((b, rows, c_out), jnp.bfloat16),
        in_specs=[
            pl.BlockSpec(memory_space=pl.ANY),
            pl.BlockSpec(memory_space=pltpu.VMEM),
        ],
        out_specs=pl.BlockSpec(memory_space=pl.ANY),
        scratch_shapes=[
            pltpu.VMEM((b, rows, c), jnp.float32),
            pltpu.VMEM((2, b, rows // NCHUNK, c_out), jnp.bfloat16),
            pltpu.VMEM((N_DEV, 2 * b, c), jnp.float32),
            pltpu.SemaphoreType.DMA((NCHUNK,)),
            pltpu.SemaphoreType.DMA((2,)),
            pltpu.SemaphoreType.DMA((N_DEV,)),
            pltpu.SemaphoreType.DMA((N_DEV,)),
        ],
        compiler_params=pltpu.CompilerParams(collective_id=0),
    )(x3, Wp)
    return out.reshape(b, h_loc, w, c_out)
